# Initial kernel scaffold; baseline (speedup 1.0000x reference)
import jax
import jax.numpy as jnp
from jax.experimental import pallas as pl


def kernel(stream, norm_w, phi_pre_w, phi_post_w, phi_res_w, b_pre, b_post, b_res, alpha_pre, alpha_post, alpha_res, swiglu_norm_w, swiglu_wd_w, swiglu_wu_w, swiglu_gate_w, swiglu_up_w, swiglu_down_w, router_w):
    raise NotImplementedError("write your pallas kernel here")


# speedup vs baseline: 1.4189x; 1.4189x over previous
import jax
import jax.numpy as jnp
from jax.experimental import pallas as pl
from jax.experimental.pallas import tpu as pltpu

D_H = 768
N_EXP = 8
N_M = 4
ND = N_M * D_H
D_F = int(D_H * 1.618)
TOP_P = 0.8
MAX_KSEL = 4
N_ACT = N_EXP - 1

TT = 256


def _dot_t(a, b, prec=None):
    return jax.lax.dot_general(
        a, b, (((1,), (1,)), ((), ())),
        preferred_element_type=jnp.float32, precision=prec)


def _moe_body(stream_ref, nw_ref, phi_ref, bias_ref, alpha_ref, swn_ref,
              wd_ref, wu_ref, wg_ref, wup_ref, wdn_ref, rw_ref,
              out_ref, gates_ref, lp_ref, gates_scr):
    e = pl.program_id(1)

    s0 = stream_ref[0]
    s1 = stream_ref[1]
    s2 = stream_ref[2]
    s3 = stream_ref[3]

    @pl.when(e == 0)
    def _router():
        xm = (s0 + s1 + s2 + s3) * 0.25
        logits = _dot_t(xm, rw_ref[...])
        m = jnp.max(logits, axis=1, keepdims=True)
        p = jnp.exp(logits - m)
        p = p / jnp.sum(p, axis=1, keepdims=True)
        colid = jax.lax.broadcasted_iota(jnp.int32, p.shape, 1)
        s_before = jnp.zeros_like(p)
        rank = jnp.zeros_like(p)
        for i in range(N_EXP):
            pi = p[:, i:i + 1]
            before = (pi > p) | ((pi == p) & (i < colid))
            bf = before.astype(jnp.float32)
            s_before = s_before + pi * bf
            rank = rank + bf
        mask = ((s_before < TOP_P) & (rank < MAX_KSEL)) | (rank == 0)
        gates = p * mask.astype(jnp.float32)
        gates_scr[...] = gates
        gates_ref[...] = gates
        logp = jnp.maximum(jnp.log(p), -10.0)
        lp_ref[...] = jnp.sum(
            logp * (gates > 0).astype(jnp.float32), axis=1, keepdims=True)
        out_ref[...] = jnp.zeros_like(out_ref)

    eidx = e + 1
    oh = (jax.lax.broadcasted_iota(jnp.int32, (1, N_EXP), 1) == eidx)
    gate_col = jnp.sum(
        gates_scr[...] * oh.astype(jnp.float32), axis=1, keepdims=True)

    @pl.when(jnp.max(gate_col) > 0.0)
    def _expert():
        streams = (s0, s1, s2, s3)
        ssq = (jnp.sum(s0 * s0, axis=1, keepdims=True)
               + jnp.sum(s1 * s1, axis=1, keepdims=True)
               + jnp.sum(s2 * s2, axis=1, keepdims=True)
               + jnp.sum(s3 * s3, axis=1, keepdims=True))
        rms = jax.lax.rsqrt(ssq * (1.0 / ND) + 1e-08)
        nwb = nw_ref[0]
        z = _dot_t(streams[0] * rms * nwb[0:1], phi_ref[0, 0])
        for n in range(1, N_M):
            z = z + _dot_t(streams[n] * rms * nwb[n:n + 1], phi_ref[0, n])
        a = alpha_ref[0]
        b = bias_ref[0]
        h_pre = jax.nn.sigmoid(z[:, 0:4] * a[:, 0:1] + b[:, 0:4])
        h_post = 2.0 * jax.nn.sigmoid(z[:, 4:8] * a[:, 1:2] + b[:, 4:8])
        mres = jnp.exp(z[:, 8:24] * a[:, 2:3] + b[:, 8:24])
        for _ in range(6):
            rsum = [jnp.sum(mres[:, 4 * i:4 * i + 4], axis=1, keepdims=True)
                    for i in range(4)]
            mres = jnp.concatenate(
                [mres[:, 4 * i:4 * i + 4] / rsum[i] for i in range(4)], axis=1)
            csum = (mres[:, 0:4] + mres[:, 4:8]
                    + mres[:, 8:12] + mres[:, 12:16])
            mres = mres / jnp.concatenate([csum] * 4, axis=1)

        h_e = (h_pre[:, 0:1] * s0 + h_pre[:, 1:2] * s1
               + h_pre[:, 2:3] * s2 + h_pre[:, 3:4] * s3)
        ssq2 = jnp.sum(h_e * h_e, axis=1, keepdims=True)
        rms2 = jax.lax.rsqrt(ssq2 * (1.0 / D_H) + 1e-08)
        h = h_e * rms2 * swn_ref[0]

        wdo = _dot_t(h, wd_ref[0])
        g = jax.nn.sigmoid(_dot_t(jax.nn.silu(wdo), wu_ref[0]))
        go = _dot_t(h, wg_ref[0])
        uo = _dot_t(h, wup_ref[0])
        out_e = g * _dot_t(jax.nn.silu(go) * uo, wdn_ref[0])

        for n in range(N_M):
            res_n = (mres[:, 4 * n:4 * n + 1] * s0
                     + mres[:, 4 * n + 1:4 * n + 2] * s1
                     + mres[:, 4 * n + 2:4 * n + 3] * s2
                     + mres[:, 4 * n + 3:4 * n + 4] * s3)
            post_n = h_post[:, n:n + 1] * out_e
            out_ref[n] += gate_col * (res_n + post_n)


def kernel(stream, norm_w, phi_pre_w, phi_post_w, phi_res_w, b_pre, b_post,
           b_res, alpha_pre, alpha_post, alpha_res, swiglu_norm_w,
           swiglu_wd_w, swiglu_wu_w, swiglu_gate_w, swiglu_up_w,
           swiglu_down_w, router_w):
    Bs, n, T, d = stream.shape
    E = router_w.shape[0]
    s3 = stream[0]
    phi_cat = jnp.concatenate([phi_pre_w, phi_post_w, phi_res_w], axis=1)
    phi_cat = jnp.transpose(phi_cat.reshape(E, 24, N_M, d), (0, 2, 1, 3))
    bias_cat = jnp.concatenate(
        [b_pre, b_post, b_res.reshape(E, N_M * N_M)], axis=1)[:, None, :]
    alpha_cat = jnp.stack([alpha_pre, alpha_post, alpha_res], axis=1)[:, None, :]
    nw3 = norm_w.reshape(E, N_M, d)
    swn3 = swiglu_norm_w[:, None, :]

    nt = T // TT
    grid = (nt, N_ACT)

    out, gates, lp = pl.pallas_call(
        _moe_body,
        grid=grid,
        in_specs=[
            pl.BlockSpec((N_M, TT, D_H), lambda tt, e: (0, tt, 0)),
            pl.BlockSpec((1, N_M, D_H), lambda tt, e: (e + 1, 0, 0)),
            pl.BlockSpec((1, N_M, 24, D_H), lambda tt, e: (e + 1, 0, 0, 0)),
            pl.BlockSpec((1, 1, 24), lambda tt, e: (e + 1, 0, 0)),
            pl.BlockSpec((1, 1, 3), lambda tt, e: (e + 1, 0, 0)),
            pl.BlockSpec((1, 1, D_H), lambda tt, e: (e + 1, 0, 0)),
            pl.BlockSpec((1, D_H, D_H), lambda tt, e: (e + 1, 0, 0)),
            pl.BlockSpec((1, D_H, D_H), lambda tt, e: (e + 1, 0, 0)),
            pl.BlockSpec((1, D_F, D_H), lambda tt, e: (e + 1, 0, 0)),
            pl.BlockSpec((1, D_F, D_H), lambda tt, e: (e + 1, 0, 0)),
            pl.BlockSpec((1, D_H, D_F), lambda tt, e: (e + 1, 0, 0)),
            pl.BlockSpec((N_EXP, D_H), lambda tt, e: (0, 0)),
        ],
        out_specs=[
            pl.BlockSpec((N_M, TT, D_H), lambda tt, e: (0, tt, 0)),
            pl.BlockSpec((TT, N_EXP), lambda tt, e: (tt, 0)),
            pl.BlockSpec((TT, 1), lambda tt, e: (tt, 0)),
        ],
        out_shape=[
            jax.ShapeDtypeStruct((N_M, T, D_H), jnp.float32),
            jax.ShapeDtypeStruct((T, N_EXP), jnp.float32),
            jax.ShapeDtypeStruct((T, 1), jnp.float32),
        ],
        scratch_shapes=[pltpu.VMEM((TT, N_EXP), jnp.float32)],
        compiler_params=pltpu.CompilerParams(
            dimension_semantics=("arbitrary", "arbitrary"),
            vmem_limit_bytes=67_000_000,
        ),
    )(s3, nw3, phi_cat, bias_cat, alpha_cat, swn3,
      swiglu_wd_w, swiglu_wu_w, swiglu_gate_w, swiglu_up_w, swiglu_down_w,
      router_w)

    return out[None], gates[None], lp.reshape(1, T)
